# baseline (speedup 1.0000x reference)
import functools

import jax
import jax.numpy as jnp
from jax import lax
from jax.experimental import pallas as pl
from jax.experimental.pallas import tpu as pltpu
from jax.experimental.pallas import tpu_sc as plsc

_N = 50000
_D = 512
_S = _N - 1
_TR = 80
_NT = _N // _TR
_NW = 32
_TPW = 20
_SCORES_PAD = _N + 16
_CH = 400
_NCH = _N // _CH
_CPW = 4
_NEG = float("-inf")

_mesh = plsc.VectorSubcoreMesh(core_axis_name="c", subcore_axis_name="s")


def _k1_body(hv_hbm, w_hbm, scores_hbm, stats_hbm,
             w_v, hb0, hb1, sc_all, stat_v, sem0, sem1, semo):
    wid = lax.axis_index("s") * 2 + lax.axis_index("c")
    iota = lax.iota(jnp.int32, 16)
    pltpu.sync_copy(w_hbm, w_v)
    wv = [w_v[pl.ds(16 * k, 16)] for k in range(32)]
    hbufs = (hb0, hb1)
    sems = (sem0, sem1)

    def tile_id(l):
        t = wid + _NW * l
        return jnp.where(t < _NT, t, 0), t

    def in_copy(l):
        t, _ = tile_id(l)
        return pltpu.make_async_copy(
            hv_hbm.at[pl.ds(t * _TR, _TR)], hbufs[l % 2], sems[l % 2])

    def out_copy(l):
        t, _ = tile_id(l)
        return pltpu.make_async_copy(
            sc_all.at[pl.ds(l * _TR, _TR)], scores_hbm.at[pl.ds(t * _TR, _TR)],
            semo)

    in_copy(0).start()
    vm = jnp.full((16,), _NEG, jnp.float32)
    vz = jnp.zeros((16,), jnp.float32)

    for l in range(_TPW):
        if l + 1 < _TPW:
            in_copy(l + 1).start()
        in_copy(l).wait()
        hb = hbufs[l % 2]
        t, traw = tile_id(l)
        base = l * _TR
        validv = jnp.full((16,), traw, jnp.int32) < _NT

        lane0 = iota == 0

        def row_pair(r2, carry, hb=hb, base=base):
            r = r2 * 2
            for j in (0, 1):
                rr = r + j
                ps = [hb[rr, pl.ds(16 * k, 16)] * wv[k] for k in range(32)]
                while len(ps) > 1:
                    ps = [ps[i] + ps[i + 1] for i in range(0, len(ps), 2)]
                plsc.store_scatter(
                    sc_all, [jnp.full((16,), base + rr, jnp.int32)],
                    jnp.full((16,), jnp.sum(ps[0])), mask=lane0)
            return carry

        lax.fori_loop(0, _TR // 2, row_pair, 0)

        for g in range(_TR // 16):
            v = sc_all[pl.ds(base + 16 * g, 16)]
            rid = jnp.full((16,), t * _TR + 16 * g, jnp.int32) + iota
            v = jnp.where(rid == _S, _NEG, v)
            sc_all[pl.ds(base + 16 * g, 16)] = v
            nm = jnp.where(validv, jnp.maximum(vm, v), vm)
            vz = vz * jnp.exp(vm - nm) + jnp.where(validv, jnp.exp(v - nm), 0.0)
            vm = nm
        out_copy(l).start()

    for l in range(_TPW):
        out_copy(l).wait()

    m_w = jnp.max(vm)
    z_w = jnp.sum(vz * jnp.exp(vm - jnp.full((16,), m_w)))
    stat_v[...] = jnp.where(iota == 0, jnp.full((16,), m_w),
                            jnp.where(iota == 1, jnp.full((16,), z_w), 0.0))
    pltpu.sync_copy(stat_v, stats_hbm.at[wid])


_k1 = functools.partial(
    pl.kernel,
    out_type=[
        jax.ShapeDtypeStruct((_SCORES_PAD,), jnp.float32),
        jax.ShapeDtypeStruct((_NW, 16), jnp.float32),
    ],
    mesh=_mesh,
    compiler_params=pltpu.CompilerParams(needs_layout_passes=False),
    scratch_types=[
        pltpu.VMEM((_D,), jnp.float32),
        pltpu.VMEM((_TR, _D), jnp.float32),
        pltpu.VMEM((_TR, _D), jnp.float32),
        pltpu.VMEM((_TPW * _TR,), jnp.float32),
        pltpu.VMEM((16,), jnp.float32),
        pltpu.SemaphoreType.DMA,
        pltpu.SemaphoreType.DMA,
        pltpu.SemaphoreType.DMA,
    ],
)(_k1_body)


def _k2_body(scores_hbm, stats_hbm, dest_hbm, probs_hbm, logp_hbm,
             st_v, dbuf, sbuf, pbuf, dsc, lbuf):
    wid = lax.axis_index("s") * 2 + lax.axis_index("c")
    iota = lax.iota(jnp.int32, 16)
    pltpu.sync_copy(stats_hbm, st_v)
    zz = jnp.zeros((16,), jnp.int32)
    oo = jnp.full((16,), 1, jnp.int32)
    m1 = plsc.load_gather(st_v, [iota, zz])
    m2 = plsc.load_gather(st_v, [iota + 16, zz])
    z1 = plsc.load_gather(st_v, [iota, oo])
    z2 = plsc.load_gather(st_v, [iota + 16, oo])
    m_g = jnp.max(jnp.maximum(m1, m2))
    mgv = jnp.full((16,), m_g)
    z_g = jnp.sum(z1 * jnp.exp(m1 - mgv) + z2 * jnp.exp(m2 - mgv))
    zgv = jnp.full((16,), z_g)
    rzv = jnp.full((16,), 1.0, jnp.float32) / zgv

    for ci in range(_CPW):
        ch = wid + _NW * ci
        ch = jnp.where(ch < _NCH, ch, 0)
        pltpu.sync_copy(scores_hbm.at[pl.ds(ch * _CH, _CH)], sbuf)
        for g in range(_CH // 16):
            v = sbuf[pl.ds(16 * g, 16)]
            pbuf[pl.ds(16 * g, 16)] = jnp.exp(v - mgv) * rzv
        pltpu.sync_copy(pbuf, probs_hbm.at[pl.ds(ch * _CH, _CH)])

    @pl.when(wid == 0)
    def _logp():
        pltpu.sync_copy(dest_hbm, dbuf)
        d = dbuf[...][0]
        base8 = (d // 8) * 8
        pltpu.sync_copy(scores_hbm.at[pl.ds(base8, 16)], dsc)
        v = dsc[...]
        rid = jnp.full((16,), base8, jnp.int32) + iota
        sd = jnp.max(jnp.where(rid == jnp.full((16,), d), v, _NEG))
        bits = plsc.bitcast(zgv, jnp.int32)
        e = ((bits >> 23) & 0xFF) - 127
        y = e.astype(jnp.float32) * jnp.float32(0.6931471805599453)
        for _ in range(4):
            y = y + zgv * jnp.exp(-y) - 1.0
        lbuf[...] = jnp.full((16,), sd) - mgv - y
        pltpu.sync_copy(lbuf, logp_hbm)


_k2 = functools.partial(
    pl.kernel,
    out_type=[
        jax.ShapeDtypeStruct((_N,), jnp.float32),
        jax.ShapeDtypeStruct((16,), jnp.float32),
    ],
    mesh=_mesh,
    compiler_params=pltpu.CompilerParams(needs_layout_passes=False),
    scratch_types=[
        pltpu.VMEM((_NW, 16), jnp.float32),
        pltpu.VMEM((16,), jnp.int32),
        pltpu.VMEM((_CH,), jnp.float32),
        pltpu.VMEM((_CH,), jnp.float32),
        pltpu.VMEM((16,), jnp.float32),
        pltpu.VMEM((16,), jnp.float32),
    ],
)(_k2_body)


def kernel(hv, W, b, dest):
    del b
    w1 = W[0, :_D]
    dest_v = jnp.full((16,), dest, jnp.int32)
    scores, stats = _k1(hv, w1)
    probs, logp = _k2(scores, stats, dest_v)
    return (probs[:_S].reshape(1, _S), logp[:1].reshape(1, 1))


# speedup vs baseline: 1.2916x; 1.2916x over previous
import functools

import jax
import jax.numpy as jnp
from jax import lax
from jax.experimental import pallas as pl
from jax.experimental.pallas import tpu as pltpu
from jax.experimental.pallas import tpu_sc as plsc

_N = 50000
_D = 512
_S = _N - 1
_TR = 80
_NT = _N // _TR
_NW = 32
_TPW = 20
_PAD = _N + 16
_NEG = float("-inf")

_mesh = plsc.VectorSubcoreMesh(core_axis_name="c", subcore_axis_name="s")


def _k1_body(hv_hbm, w_hbm, scores_hbm,
             w_v, hb0, hb1, sc_all, sem0, sem1, semo):
    wid = lax.axis_index("s") * 2 + lax.axis_index("c")
    iota = lax.iota(jnp.int32, 16)
    lane0 = iota == 0
    pltpu.sync_copy(w_hbm, w_v)
    wv = [w_v[pl.ds(16 * k, 16)] for k in range(32)]
    hbufs = (hb0, hb1)
    sems = (sem0, sem1)

    def tile_id(l):
        t = wid + _NW * l
        return jnp.where(t < _NT, t, 0)

    def in_copy(l):
        t = tile_id(l)
        return pltpu.make_async_copy(
            hv_hbm.at[pl.ds(t * _TR, _TR)], hbufs[l % 2], sems[l % 2])

    def out_copy(l):
        t = tile_id(l)
        return pltpu.make_async_copy(
            sc_all.at[pl.ds(l * _TR, _TR)], scores_hbm.at[pl.ds(t * _TR, _TR)],
            semo)

    in_copy(0).start()

    for l in range(_TPW):
        if l + 1 < _TPW:
            in_copy(l + 1).start()
        in_copy(l).wait()
        hb = hbufs[l % 2]
        base = l * _TR

        @plsc.parallel_loop(0, _TR, 1, unroll=4)
        def _row(rr, hb=hb, base=base):
            ps = [hb[rr, pl.ds(16 * k, 16)] * wv[k] for k in range(32)]
            while len(ps) > 1:
                ps = [ps[i] + ps[i + 1] for i in range(0, len(ps), 2)]
            plsc.store_scatter(
                sc_all, [jnp.full((16,), base + rr, jnp.int32)],
                jnp.full((16,), jnp.sum(ps[0])), mask=lane0)

        out_copy(l).start()

    for l in range(_TPW):
        out_copy(l).wait()


_k1 = functools.partial(
    pl.kernel,
    out_type=[jax.ShapeDtypeStruct((_PAD,), jnp.float32)],
    mesh=_mesh,
    compiler_params=pltpu.CompilerParams(needs_layout_passes=False),
    scratch_types=[
        pltpu.VMEM((_D,), jnp.float32),
        pltpu.VMEM((_TR, _D), jnp.float32),
        pltpu.VMEM((_TR, _D), jnp.float32),
        pltpu.VMEM((_TPW * _TR,), jnp.float32),
        pltpu.SemaphoreType.DMA,
        pltpu.SemaphoreType.DMA,
        pltpu.SemaphoreType.DMA,
    ],
)(_k1_body)


def _ep_body(dest_ref, sc_ref, probs_ref, logp_ref):
    s = sc_ref[...]
    col = lax.broadcasted_iota(jnp.int32, (1, _PAD), 1)
    s = jnp.where(col >= _S, _NEG, s)
    m = jnp.max(s)
    e = jnp.exp(s - m)
    z = jnp.sum(e)
    probs_ref[...] = e[:, :_S] * (1.0 / z)
    d = dest_ref[0]
    sd = jnp.max(jnp.where(col == d, s, _NEG))
    logp_ref[...] = jnp.broadcast_to(sd - m - jnp.log(z), (1, 1))


def kernel(hv, W, b, dest):
    del b
    w1 = W[0, :_D]
    dest_arr = jnp.asarray(dest, dtype=jnp.int32).reshape((1,))
    (scores,) = _k1(hv, w1)
    probs, logp = pl.pallas_call(
        _ep_body,
        in_specs=[
            pl.BlockSpec(memory_space=pltpu.SMEM),
            pl.BlockSpec((1, _PAD), lambda: (0, 0)),
        ],
        out_specs=[
            pl.BlockSpec((1, _S), lambda: (0, 0)),
            pl.BlockSpec((1, 1), lambda: (0, 0)),
        ],
        out_shape=[
            jax.ShapeDtypeStruct((1, _S), jnp.float32),
            jax.ShapeDtypeStruct((1, 1), jnp.float32),
        ],
    )(dest_arr, scores.reshape(1, _PAD))
    return (probs, logp)
